# baseline (speedup 1.0000x reference)
import functools

import jax
import jax.numpy as jnp
from jax.experimental import pallas as pl
from jax.experimental.pallas import tpu as pltpu

_VAR0 = 0.1
_VAR1 = 0.2
_TOP_K = 200
_P = 20000
_PP = 20480
_ROWS = 160
_LANES = 128
_NEG = -jnp.inf
_G = 4


def _prep_kernel(ct_ref, conf_ref, loc_ref, prior_ref, scores_ref, pts_ref):
    conf_t = ct_ref[0]
    conf = conf_ref[0]
    mx = jnp.max(conf, axis=0, keepdims=True)
    e = jnp.exp(conf - mx)
    z = jnp.sum(e, axis=0, keepdims=True)
    probs = e / z
    fg = probs[1:, :]
    pos = jax.lax.broadcasted_iota(jnp.int32, (20, _PP), 1)
    valid = (fg >= conf_t) & (pos < _P)
    scores_ref[0] = jnp.where(valid, fg, _NEG)

    l = loc_ref[0]
    lx, ly, lw, lh = l[0:1], l[1:2], l[2:3], l[3:4]
    pcx, pcy, pw, ph = (prior_ref[0:1], prior_ref[1:2],
                        prior_ref[2:3], prior_ref[3:4])
    cx = pcx + lx * (_VAR0) * pw
    cy = pcy + ly * (_VAR0) * ph
    w = pw * jnp.exp(lw * _VAR1)
    h = ph * jnp.exp(lh * _VAR1)
    x1 = cx - w * 0.5
    y1 = cy - h * 0.5
    x2 = cx + w * 0.5
    y2 = cy + h * 0.5
    pts_ref[0, 0:1, :] = x1
    pts_ref[0, 1:2, :] = y1
    pts_ref[0, 2:3, :] = x2
    pts_ref[0, 3:4, :] = y2
    pts_ref[0, 4:5, :] = cx
    pts_ref[0, 5:6, :] = cy
    pts_ref[0, 6:7, :] = w
    pts_ref[0, 7:8, :] = h


def _nms_kernel(nt_ref, scores_ref, pts_ref, out_ref,
                sc_ref, kx1_ref, ky1_ref, kx2_ref, ky2_ref, ka_ref):
    nms_t = nt_ref[0]
    out_ref[0] = jnp.zeros((_G, _TOP_K, 8), jnp.float32)
    sc_ref[...] = scores_ref[0, 0]
    z = jnp.zeros((_G, 8, _LANES), jnp.float32)
    kx1_ref[...] = z
    ky1_ref[...] = z
    kx2_ref[...] = z
    ky2_ref[...] = z
    ka_ref[...] = z

    iota = (jax.lax.broadcasted_iota(jnp.int32, (_ROWS, _LANES), 0) * _LANES
            + jax.lax.broadcasted_iota(jnp.int32, (_ROWS, _LANES), 1))
    lane = jax.lax.broadcasted_iota(jnp.int32, (1, _LANES), 1)
    row8 = jax.lax.broadcasted_iota(jnp.int32, (1, 8), 1)
    iota8 = (jax.lax.broadcasted_iota(jnp.int32, (8, _LANES), 0) * _LANES
             + jax.lax.broadcasted_iota(jnp.int32, (8, _LANES), 1))

    def cond(carry):
        alive = False
        for g in range(_G):
            k, done = carry[g]
            alive = alive | ((k < _TOP_K) & jnp.logical_not(done))
        return alive

    def body(carry):
        new_carry = []
        for g in range(_G):
            k, done = carry[g]
            s = sc_ref[g]
            m = jnp.max(s)
            sel = m > _NEG
            idx = jnp.min(jnp.where(s == m, iota, jnp.int32(2 ** 30)))
            r = idx // _LANES
            c = idx % _LANES
            lmask = lane == c

            def pick(j):
                row = pts_ref[0, j, pl.ds(r, 1), :]
                return jnp.sum(jnp.where(lmask, row, 0.0))

            x1s = pick(0)
            y1s = pick(1)
            x2s = pick(2)
            y2s = pick(3)
            cxs = pick(4)
            cys = pick(5)
            ws = pick(6)
            hs = pick(7)

            iw = jnp.maximum(jnp.minimum(x2s, kx2_ref[g])
                             - jnp.maximum(x1s, kx1_ref[g]), 0.0)
            ih = jnp.maximum(jnp.minimum(y2s, ky2_ref[g])
                             - jnp.maximum(y1s, ky1_ref[g]), 0.0)
            inter = iw * ih
            area1 = (jnp.maximum(x2s - x1s, 0.0)
                     * jnp.maximum(y2s - y1s, 0.0))
            union = area1 + ka_ref[g] - inter
            supp_any = jnp.any(inter > nms_t * jnp.maximum(union, 1e-12))
            keep = sel & jnp.logical_not(supp_any) & (k < _TOP_K)

            @pl.when(sel)
            def _(g=g, r=r, lmask=lmask):
                rowv = sc_ref[g, pl.ds(r, 1), :]
                sc_ref[g, pl.ds(r, 1), :] = jnp.where(lmask, _NEG, rowv)

            @pl.when(keep)
            def _(g=g, k=k, x1s=x1s, y1s=y1s, x2s=x2s, y2s=y2s,
                  cxs=cxs, cys=cys, ws=ws, hs=hs, m=m, area1=area1):
                onehot = iota8 == k
                kx1_ref[g] = jnp.where(onehot, x1s, kx1_ref[g])
                ky1_ref[g] = jnp.where(onehot, y1s, ky1_ref[g])
                kx2_ref[g] = jnp.where(onehot, x2s, kx2_ref[g])
                ky2_ref[g] = jnp.where(onehot, y2s, ky2_ref[g])
                ka_ref[g] = jnp.where(onehot, area1, ka_ref[g])
                row_out = jnp.where(row8 == 0, m,
                          jnp.where(row8 == 1, cxs,
                          jnp.where(row8 == 2, cys,
                          jnp.where(row8 == 3, ws,
                          jnp.where(row8 == 4, hs, 0.0)))))
                out_ref[0, g, pl.ds(k, 1), :] = row_out

            k2 = k + jnp.where(keep, 1, 0)
            new_carry.append((k2, jnp.logical_not(sel)))
        return tuple(new_carry)

    init = tuple((jnp.int32(0), False) for _ in range(_G))
    jax.lax.while_loop(cond, body, init)


@jax.jit
def kernel(loc_data, conf_data, prior_data, conf_thresh, nms_thresh):
    B = loc_data.shape[0]
    C = conf_data.shape[1] - 1

    conf_p = jnp.pad(conf_data, ((0, 0), (0, 0), (0, _PP - _P)))
    loc_p = jnp.pad(loc_data, ((0, 0), (0, 0), (0, _PP - _P)))
    prior_p = jnp.pad(prior_data.T, ((0, 0), (0, _PP - _P)))

    scores, pts = pl.pallas_call(
        _prep_kernel,
        grid=(B,),
        in_specs=[
            pl.BlockSpec(memory_space=pltpu.SMEM),
            pl.BlockSpec((1, C + 1, _PP), lambda b: (b, 0, 0)),
            pl.BlockSpec((1, 4, _PP), lambda b: (b, 0, 0)),
            pl.BlockSpec((4, _PP), lambda b: (0, 0)),
        ],
        out_specs=[
            pl.BlockSpec((1, C, _PP), lambda b: (b, 0, 0)),
            pl.BlockSpec((1, 8, _PP), lambda b: (b, 0, 0)),
        ],
        out_shape=[
            jax.ShapeDtypeStruct((B, C, _PP), jnp.float32),
            jax.ShapeDtypeStruct((B, 8, _PP), jnp.float32),
        ],
    )(conf_thresh.reshape(1), conf_p, loc_p, prior_p)

    scores5 = scores.reshape(B, C // _G, _G, _ROWS, _LANES)
    pts4 = pts.reshape(B, 8, _ROWS, _LANES)

    out = pl.pallas_call(
        _nms_kernel,
        grid=(B, C // _G),
        in_specs=[
            pl.BlockSpec(memory_space=pltpu.SMEM),
            pl.BlockSpec((1, 1, _G, _ROWS, _LANES),
                         lambda b, cg: (b, cg, 0, 0, 0)),
            pl.BlockSpec((1, 8, _ROWS, _LANES), lambda b, cg: (b, 0, 0, 0)),
        ],
        out_specs=pl.BlockSpec((1, _G, _TOP_K, 8),
                               lambda b, cg: (b * (20 // _G) + cg, 0, 0, 0)),
        out_shape=jax.ShapeDtypeStruct((B * (20 // _G), _G, _TOP_K, 8),
                                       jnp.float32),
        scratch_shapes=[
            pltpu.VMEM((_G, _ROWS, _LANES), jnp.float32),
            pltpu.VMEM((_G, 8, _LANES), jnp.float32),
            pltpu.VMEM((_G, 8, _LANES), jnp.float32),
            pltpu.VMEM((_G, 8, _LANES), jnp.float32),
            pltpu.VMEM((_G, 8, _LANES), jnp.float32),
            pltpu.VMEM((_G, 8, _LANES), jnp.float32),
        ],
    )(nms_thresh.reshape(1), scores5, pts4)

    out = out.reshape(B, C, _TOP_K, 8)
    return out[..., :5]


# speedup vs baseline: 47.8853x; 47.8853x over previous
import functools

import jax
import jax.numpy as jnp
from jax import lax
from jax.experimental import pallas as pl
from jax.experimental.pallas import tpu as pltpu
from jax.experimental.pallas import tpu_sc as plsc

_VAR0 = 0.1
_VAR1 = 0.2
_TOP_K = 200
_P = 20000
_PP = 20480
_NSL = 80
_NCH = _PP // 16
_NL1 = _NCH // 16
_NL2 = 5
_KPAD = 224
_OUTW = _TOP_K * 8 + 16


def _prep_kernel(ct_ref, conf_ref, loc_ref, prior_ref, scores_ref, pts_ref):
    conf_t = ct_ref[0]
    conf = conf_ref[0]
    mx = jnp.max(conf, axis=0, keepdims=True)
    e = jnp.exp(conf - mx)
    z = jnp.sum(e, axis=0, keepdims=True)
    probs = e / z
    fg = probs[1:, :]
    pos = jax.lax.broadcasted_iota(jnp.int32, (20, _PP), 1)
    valid = (fg >= conf_t) & (pos < _P)
    scores_ref[0] = jnp.where(valid, fg, 0.0)

    l = loc_ref[0]
    lx, ly, lw, lh = l[0:1], l[1:2], l[2:3], l[3:4]
    pcx, pcy, pw, ph = (prior_ref[0:1], prior_ref[1:2],
                        prior_ref[2:3], prior_ref[3:4])
    cx = pcx + lx * (_VAR0) * pw
    cy = pcy + ly * (_VAR0) * ph
    w = pw * jnp.exp(lw * _VAR1)
    h = ph * jnp.exp(lh * _VAR1)
    pts_ref[0, 0:1, :] = cx - w * 0.5
    pts_ref[0, 1:2, :] = cy - h * 0.5
    pts_ref[0, 2:3, :] = cx + w * 0.5
    pts_ref[0, 3:4, :] = cy + h * 0.5


def _sc_nms(nt_hbm, scores_hbm, pts_hbm, out_hbm,
            thr_v, s_v, x1_v, y1_v, x2_v, y2_v,
            l1_v, l2_v, kx1_v, ky1_v, kx2_v, ky2_v, ka_v, out_v):
    nc = 2
    wid = lax.axis_index("s") * nc + lax.axis_index("c")
    lane = lax.iota(jnp.int32, 16)
    zero16 = jnp.zeros((16,), jnp.float32)
    eps16 = jnp.full((16,), 1e-12, jnp.float32)
    last_mask = lane == 15

    pltpu.sync_copy(nt_hbm, thr_v)
    thrv = thr_v[...]

    for si in range(3):
        sl = si * 32 + wid

        @pl.when(sl < _NSL)
        def _process(sl=sl):
            bi = sl // 20
            pltpu.sync_copy(scores_hbm.at[sl], s_v)
            pltpu.sync_copy(pts_hbm.at[bi, 0], x1_v)
            pltpu.sync_copy(pts_hbm.at[bi, 1], y1_v)
            pltpu.sync_copy(pts_hbm.at[bi, 2], x2_v)
            pltpu.sync_copy(pts_hbm.at[bi, 3], y2_v)

            def _zero_k(i, _):
                kx1_v[pl.ds(i * 16, 16)] = zero16
                ky1_v[pl.ds(i * 16, 16)] = zero16
                kx2_v[pl.ds(i * 16, 16)] = zero16
                ky2_v[pl.ds(i * 16, 16)] = zero16
                ka_v[pl.ds(i * 16, 16)] = zero16
                return 0
            lax.fori_loop(0, _KPAD // 16, _zero_k, 0)

            def _zero_o(i, _):
                out_v[pl.ds(i * 16, 16)] = zero16
                return 0
            lax.fori_loop(0, _OUTW // 16, _zero_o, 0)

            def _build_l1(ch, _):
                v = s_v[pl.ds(ch * 16, 16)]
                cm = plsc.cummax(v)
                idxv = jnp.full((16,), ch, jnp.int32)
                plsc.store_scatter(l1_v, [idxv], cm, mask=last_mask)
                return 0
            lax.fori_loop(0, _NCH, _build_l1, 0)

            def _build_l2(g, _):
                v = l1_v[pl.ds(g * 16, 16)]
                cm = plsc.cummax(v)
                idxv = jnp.full((16,), g, jnp.int32)
                plsc.store_scatter(l2_v, [idxv], cm, mask=last_mask)
                return 0
            lax.fori_loop(0, _NL1, _build_l2, 0)

            def cond(carry):
                k, done = carry
                return (k < _TOP_K) & jnp.logical_not(done)

            def body(carry):
                k, done = carry
                t0 = l2_v[pl.ds(0, 16)]
                t1 = l2_v[pl.ds(16, 16)]
                t2 = l2_v[pl.ds(32, 16)]
                t3 = l2_v[pl.ds(48, 16)]
                t4 = l2_v[pl.ds(64, 16)]
                mv = jnp.maximum(jnp.maximum(jnp.maximum(t0, t1),
                                             jnp.maximum(t2, t3)), t4)
                m = jnp.max(mv)
                sel = m > 0.0

                c2v = jnp.full((16,), jnp.int32(2 ** 30))
                for i, tv in ((4, t4), (3, t3), (2, t2), (1, t1), (0, t0)):
                    mk = tv == m
                    cnt = plsc.all_reduce_population_count(mk)
                    ffs = plsc.all_reduce_ffs(mk)
                    c2v = jnp.where(cnt > 0, i * 16 + ffs, c2v)
                c2 = jnp.max(jnp.where(lane == 0, c2v, 0))

                v1 = l1_v[pl.ds(c2 * 16, 16)]
                chv = c2 * 16 + plsc.all_reduce_ffs(v1 == m)
                ch = jnp.max(jnp.where(lane == 0, chv, 0))

                sv = s_v[pl.ds(ch * 16, 16)]
                ffs_e = plsc.all_reduce_ffs(sv == m)
                idxv = ch * 16 + ffs_e

                sv2 = jnp.where(lane == ffs_e, 0.0, sv)
                s_v[pl.ds(ch * 16, 16)] = sv2
                cm1 = jnp.max(sv2)
                nl1 = jnp.where(lane == (ch - c2 * 16), cm1, v1)
                l1_v[pl.ds(c2 * 16, 16)] = nl1
                cm2 = jnp.max(nl1)
                g2 = c2 // 16
                l2c = l2_v[pl.ds(g2 * 16, 16)]
                l2_v[pl.ds(g2 * 16, 16)] = jnp.where(
                    lane == (c2 - g2 * 16), cm2, l2c)

                cx1 = plsc.load_gather(x1_v, [idxv])
                cy1 = plsc.load_gather(y1_v, [idxv])
                cx2 = plsc.load_gather(x2_v, [idxv])
                cy2 = plsc.load_gather(y2_v, [idxv])
                a1 = (jnp.maximum(cx2 - cx1, zero16)
                      * jnp.maximum(cy2 - cy1, zero16))

                sup = lane < 0
                for kv in range(_KPAD // 16):
                    kx1 = kx1_v[pl.ds(kv * 16, 16)]
                    ky1 = ky1_v[pl.ds(kv * 16, 16)]
                    kx2 = kx2_v[pl.ds(kv * 16, 16)]
                    ky2 = ky2_v[pl.ds(kv * 16, 16)]
                    ka = ka_v[pl.ds(kv * 16, 16)]
                    iw = jnp.maximum(jnp.minimum(cx2, kx2)
                                     - jnp.maximum(cx1, kx1), zero16)
                    ih = jnp.maximum(jnp.minimum(cy2, ky2)
                                     - jnp.maximum(cy1, ky1), zero16)
                    inter = iw * ih
                    un = a1 + ka - inter
                    sup = sup | (inter > thrv * jnp.maximum(un, eps16))
                nsup = plsc.all_reduce_population_count(sup)
                supp_any = jnp.max(nsup) > 0
                keep = sel & jnp.logical_not(supp_any)

                @pl.when(keep)
                def _():
                    kb = (k // 16) * 16
                    klane = k - kb
                    kc = kx1_v[pl.ds(kb, 16)]
                    kx1_v[pl.ds(kb, 16)] = jnp.where(lane == klane, cx1, kc)
                    kc = ky1_v[pl.ds(kb, 16)]
                    ky1_v[pl.ds(kb, 16)] = jnp.where(lane == klane, cy1, kc)
                    kc = kx2_v[pl.ds(kb, 16)]
                    kx2_v[pl.ds(kb, 16)] = jnp.where(lane == klane, cx2, kc)
                    kc = ky2_v[pl.ds(kb, 16)]
                    ky2_v[pl.ds(kb, 16)] = jnp.where(lane == klane, cy2, kc)
                    kc = ka_v[pl.ds(kb, 16)]
                    ka_v[pl.ds(kb, 16)] = jnp.where(lane == klane, a1, kc)
                    ocx = (cx1 + cx2) * 0.5
                    ocy = (cy1 + cy2) * 0.5
                    ow = cx2 - cx1
                    oh = cy2 - cy1
                    row = jnp.where(lane == 0, m,
                          jnp.where(lane == 1, ocx,
                          jnp.where(lane == 2, ocy,
                          jnp.where(lane == 3, ow,
                          jnp.where(lane == 4, oh, zero16)))))
                    out_v[pl.ds(k * 8, 16)] = row

                k2 = k + jnp.where(keep, 1, 0)
                return (k2, jnp.logical_not(sel))

            lax.while_loop(cond, body, (jnp.int32(0), False))
            pltpu.sync_copy(out_v, out_hbm.at[sl])


@jax.jit
def kernel(loc_data, conf_data, prior_data, conf_thresh, nms_thresh):
    B = loc_data.shape[0]
    C = conf_data.shape[1] - 1

    conf_p = jnp.pad(conf_data, ((0, 0), (0, 0), (0, _PP - _P)))
    loc_p = jnp.pad(loc_data, ((0, 0), (0, 0), (0, _PP - _P)))
    prior_p = jnp.pad(prior_data.T, ((0, 0), (0, _PP - _P)))

    scores, pts = pl.pallas_call(
        _prep_kernel,
        grid=(B,),
        in_specs=[
            pl.BlockSpec(memory_space=pltpu.SMEM),
            pl.BlockSpec((1, C + 1, _PP), lambda b: (b, 0, 0)),
            pl.BlockSpec((1, 4, _PP), lambda b: (b, 0, 0)),
            pl.BlockSpec((4, _PP), lambda b: (0, 0)),
        ],
        out_specs=[
            pl.BlockSpec((1, C, _PP), lambda b: (b, 0, 0)),
            pl.BlockSpec((1, 4, _PP), lambda b: (b, 0, 0)),
        ],
        out_shape=[
            jax.ShapeDtypeStruct((B, C, _PP), jnp.float32),
            jax.ShapeDtypeStruct((B, 4, _PP), jnp.float32),
        ],
    )(conf_thresh.reshape(1), conf_p, loc_p, prior_p)

    scores2 = scores.reshape(_NSL, _PP)
    ntv = jnp.broadcast_to(nms_thresh, (16,)).astype(jnp.float32)

    mesh = plsc.VectorSubcoreMesh(core_axis_name="c", subcore_axis_name="s")
    out = pl.kernel(
        _sc_nms,
        out_type=jax.ShapeDtypeStruct((_NSL, _OUTW), jnp.float32),
        mesh=mesh,
        compiler_params=pltpu.CompilerParams(needs_layout_passes=False),
        scratch_types=[
            pltpu.VMEM((16,), jnp.float32),
            pltpu.VMEM((_PP,), jnp.float32),
            pltpu.VMEM((_PP,), jnp.float32),
            pltpu.VMEM((_PP,), jnp.float32),
            pltpu.VMEM((_PP,), jnp.float32),
            pltpu.VMEM((_PP,), jnp.float32),
            pltpu.VMEM((_NCH,), jnp.float32),
            pltpu.VMEM((_NL1,), jnp.float32),
            pltpu.VMEM((_KPAD,), jnp.float32),
            pltpu.VMEM((_KPAD,), jnp.float32),
            pltpu.VMEM((_KPAD,), jnp.float32),
            pltpu.VMEM((_KPAD,), jnp.float32),
            pltpu.VMEM((_KPAD,), jnp.float32),
            pltpu.VMEM((_OUTW,), jnp.float32),
        ],
    )(ntv, scores2, pts)

    out = out[:, :_TOP_K * 8].reshape(B, C, _TOP_K, 8)
    return out[..., :5]
